# baseline (speedup 1.0000x reference)
import functools

import jax
import jax.numpy as jnp
from jax import lax
from jax.experimental import pallas as pl
from jax.experimental.pallas import tpu as pltpu
from jax.experimental.pallas import tpu_sc as plsc

_B = 8
_N = 20000
_M = 64
_NPAD = 20480
_NW = 32
_QUARTERS = 4
_CHUNK = _NPAD // _QUARTERS
_NBLK = 256
_BLOCKS = _CHUNK // _NBLK
_BUFCAP = _NBLK * _M
_ROW = _BLOCKS * _BUFCAP
_THRESH = 0.5


def _sc_body(boxes_hbm, gt_hbm, no_hbm, pos_out, neg_out, cnt_out, z_out,
             boxv, pxy, gtv, gxy, nov, pos_buf, neg_buf, tmpv):
    wid = lax.axis_index("s") * 2 + lax.axis_index("c")
    b = wid // _QUARTERS
    q = wid % _QUARTERS
    n0 = q * _CHUNK

    for c in range(4):
        pltpu.sync_copy(boxes_hbm.at[b, c, pl.ds(n0, _CHUNK)], boxv.at[c])
    pltpu.sync_copy(gt_hbm.at[b], gtv)
    pltpu.sync_copy(no_hbm.at[b], nov)
    num_obj = nov[...]

    lane = lax.iota(jnp.int32, 16)

    for v in range(_M // 16):
        s = pl.ds(16 * v, 16)
        gcx = gtv[0, s]
        gcy = gtv[1, s]
        gw = gtv[2, s]
        gh = gtv[3, s]
        gxy[0, s] = gcx - gw * 0.5
        gxy[1, s] = gcy - gh * 0.5
        gxy[2, s] = gcx + gw * 0.5
        gxy[3, s] = gcy + gh * 0.5
        gxy[4, s] = gw * gh

    def _cvt(i, _):
        s = pl.ds(16 * i, 16)
        cx = boxv[0, s]
        cy = boxv[1, s]
        w = boxv[2, s]
        h = boxv[3, s]
        pxy[0, s] = cx - w * 0.5
        pxy[1, s] = cy - h * 0.5
        pxy[2, s] = cx + w * 0.5
        pxy[3, s] = cy + h * 0.5
        pxy[4, s] = w * h
        return 0

    lax.fori_loop(0, _CHUNK // 16, _cvt, 0)

    def _block(blk, totals):
        def _group(g, counts):
            pc, nc = counts
            t0 = blk * _NBLK + g * 16
            sg = pl.ds(t0, 16)
            x1v = pxy[0, sg]
            y1v = pxy[1, sg]
            x2v = pxy[2, sg]
            y2v = pxy[3, sg]
            pav = pxy[4, sg]
            for k in range(16):
                x1 = x1v[k]
                y1 = y1v[k]
                x2 = x2v[k]
                y2 = y2v[k]
                pa = pav[k]
                n_glob = n0 + t0 + k
                row_ok = n_glob < _N
                base = n_glob * _M
                for v in range(_M // 16):
                    s = pl.ds(16 * v, 16)
                    iw = jnp.maximum(
                        jnp.minimum(x2, gxy[2, s]) - jnp.maximum(x1, gxy[0, s]),
                        0.0)
                    ih = jnp.maximum(
                        jnp.minimum(y2, gxy[3, s]) - jnp.maximum(y1, gxy[1, s]),
                        0.0)
                    inter = iw * ih
                    iou = inter / (pa + gxy[4, s] - inter)
                    col = lane + (16 * v)
                    valid = (col < num_obj) & row_ok
                    posm = (iou > _THRESH) & valid
                    negm = (iou < _THRESH) & valid
                    fi = base + col
                    pprefix = plsc.cumsum(posm.astype(jnp.int32))
                    plsc.store_scatter(pos_buf, [pc + pprefix - 1], fi,
                                       mask=posm)
                    pc = pc + pprefix[15]
                    nprefix = plsc.cumsum(negm.astype(jnp.int32))
                    plsc.store_scatter(neg_buf, [nc + nprefix - 1], fi,
                                       mask=negm)
                    nc = nc + nprefix[15]
            return pc, nc

        pc, nc = lax.fori_loop(0, _NBLK // 16, _group,
                               (jnp.int32(0), jnp.int32(0)))
        off = blk * _BUFCAP
        pltpu.sync_copy(pos_buf.at[pl.ds(0, _BUFCAP)],
                        pos_out.at[wid, pl.ds(off, _BUFCAP)])
        pltpu.sync_copy(neg_buf.at[pl.ds(0, _BUFCAP)],
                        neg_out.at[wid, pl.ds(off, _BUFCAP)])
        return totals[0] + pc, totals[1] + nc

    pos_tot, neg_tot = lax.fori_loop(
        0, _BLOCKS, _block, (jnp.int32(0), jnp.int32(0)))

    tmpv[...] = jnp.where(lane == 0, pos_tot,
                          jnp.where(lane == 1, neg_tot, 0))
    pltpu.sync_copy(tmpv, cnt_out.at[wid])

    @pl.when(wid == 0)
    def _():
        pxy[0, pl.ds(0, 16)] = lane.astype(jnp.float32) * 0.0
        pltpu.sync_copy(pxy.at[0, pl.ds(0, 16)], z_out)


@functools.partial(jax.jit, static_argnums=())
def _run_sc(boxes_t, gt_t, num_obj):
    mesh = plsc.VectorSubcoreMesh(core_axis_name="c", subcore_axis_name="s")
    kern = functools.partial(
        pl.kernel,
        mesh=mesh,
        compiler_params=pltpu.CompilerParams(needs_layout_passes=False),
        out_type=[
            jax.ShapeDtypeStruct((_NW, _ROW), jnp.int32),
            jax.ShapeDtypeStruct((_NW, _ROW), jnp.int32),
            jax.ShapeDtypeStruct((_NW, 16), jnp.int32),
            jax.ShapeDtypeStruct((16,), jnp.float32),
        ],
        scratch_types=[
            pltpu.VMEM((4, _CHUNK), jnp.float32),
            pltpu.VMEM((5, _CHUNK), jnp.float32),
            pltpu.VMEM((4, _M), jnp.float32),
            pltpu.VMEM((5, _M), jnp.float32),
            pltpu.VMEM((16,), jnp.int32),
            pltpu.VMEM((_BUFCAP + 16,), jnp.int32),
            pltpu.VMEM((_BUFCAP + 16,), jnp.int32),
            pltpu.VMEM((16,), jnp.int32),
        ],
    )(_sc_body)
    return kern(boxes_t, gt_t, num_obj)


def kernel(threshhold, batch_boxes, batch_classes, batch_gt, batch_num_objects):
    del threshhold, batch_classes
    boxes_p = jnp.pad(batch_boxes, ((0, 0), (0, _NPAD - _N), (0, 0)))
    boxes_t = jnp.transpose(boxes_p, (0, 2, 1))
    gt_t = jnp.transpose(batch_gt, (0, 2, 1))
    num_obj = jnp.broadcast_to(
        batch_num_objects.astype(jnp.int32)[:, None], (_B, 16))
    _pos, _neg, _cnt, z = _run_sc(boxes_t, gt_t, num_obj)
    return z[0]


# speedup vs baseline: 1.3365x; 1.3365x over previous
import functools

import jax
import jax.numpy as jnp
from jax import lax
from jax.experimental import pallas as pl
from jax.experimental.pallas import tpu as pltpu
from jax.experimental.pallas import tpu_sc as plsc

_B = 8
_N = 20000
_M = 64
_NPAD = 20480
_NW = 32
_QUARTERS = 4
_CHUNK = _NPAD // _QUARTERS
_NBLK = 512
_BLOCKS = _CHUNK // _NBLK
_BUFCAP = _NBLK * _M
_ROW = _BLOCKS * _BUFCAP
_THRESH = 0.5


def _sc_body(boxes_hbm, gt_hbm, no_hbm, pos_out, neg_out, cnt_out, z_out,
             boxv, pxy, gtv, gxy, nov, pos_buf, neg_buf, tmpv):
    wid = lax.axis_index("s") * 2 + lax.axis_index("c")
    b = wid // _QUARTERS
    q = wid % _QUARTERS
    n0 = q * _CHUNK

    for c in range(4):
        pltpu.sync_copy(boxes_hbm.at[b, c, pl.ds(n0, _CHUNK)], boxv.at[c])
    pltpu.sync_copy(gt_hbm.at[b], gtv)
    pltpu.sync_copy(no_hbm.at[b], nov)
    num_obj = nov[...]

    lane = lax.iota(jnp.int32, 16)

    for v in range(_M // 16):
        s = pl.ds(16 * v, 16)
        gcx = gtv[0, s]
        gcy = gtv[1, s]
        gw = gtv[2, s]
        gh = gtv[3, s]
        gxy[0, s] = gcx - gw * 0.5
        gxy[1, s] = gcy - gh * 0.5
        gxy[2, s] = gcx + gw * 0.5
        gxy[3, s] = gcy + gh * 0.5
        gxy[4, s] = gw * gh

    def _cvt(i, _):
        s = pl.ds(16 * i, 16)
        cx = boxv[0, s]
        cy = boxv[1, s]
        w = boxv[2, s]
        h = boxv[3, s]
        pxy[0, s] = cx - w * 0.5
        pxy[1, s] = cy - h * 0.5
        pxy[2, s] = cx + w * 0.5
        pxy[3, s] = cy + h * 0.5
        pxy[4, s] = w * h
        return 0

    lax.fori_loop(0, _CHUNK // 16, _cvt, 0)

    def _block(blk, totals):
        def _group(g, counts):
            pc, nc = counts
            t0 = blk * _NBLK + g * 16
            sg = pl.ds(t0, 16)
            x1v = pxy[0, sg]
            y1v = pxy[1, sg]
            x2v = pxy[2, sg]
            y2v = pxy[3, sg]
            pav = pxy[4, sg]
            for k in range(16):
                x1 = x1v[k]
                y1 = y1v[k]
                x2 = x2v[k]
                y2 = y2v[k]
                pa = pav[k]
                n_glob = n0 + t0 + k
                row_ok = n_glob < _N
                base = n_glob * _M
                for v in range(_M // 16):
                    s = pl.ds(16 * v, 16)
                    iw = jnp.maximum(
                        jnp.minimum(x2, gxy[2, s]) - jnp.maximum(x1, gxy[0, s]),
                        0.0)
                    ih = jnp.maximum(
                        jnp.minimum(y2, gxy[3, s]) - jnp.maximum(y1, gxy[1, s]),
                        0.0)
                    inter = iw * ih
                    union = pa + gxy[4, s] - inter
                    inter2 = inter + inter
                    col = lane + (16 * v)
                    valid = (col < num_obj) & row_ok
                    posm = (inter2 > union) & valid
                    negm = (inter2 < union) & valid
                    fi = base + col
                    pprefix = plsc.cumsum(posm.astype(jnp.int32))
                    plsc.store_scatter(pos_buf, [pc + pprefix - 1], fi,
                                       mask=posm)
                    pc = pc + plsc.all_reduce_population_count(posm)
                    nprefix = plsc.cumsum(negm.astype(jnp.int32))
                    plsc.store_scatter(neg_buf, [nc + nprefix - 1], fi,
                                       mask=negm)
                    nc = nc + plsc.all_reduce_population_count(negm)
            return pc, nc

        pc, nc = lax.fori_loop(0, _NBLK // 16, _group,
                               (lane * 0, lane * 0))
        off = blk * _BUFCAP
        pltpu.sync_copy(pos_buf.at[pl.ds(0, _BUFCAP)],
                        pos_out.at[wid, pl.ds(off, _BUFCAP)])
        pltpu.sync_copy(neg_buf.at[pl.ds(0, _BUFCAP)],
                        neg_out.at[wid, pl.ds(off, _BUFCAP)])
        return totals[0] + pc, totals[1] + nc

    pos_tot, neg_tot = lax.fori_loop(
        0, _BLOCKS, _block, (lane * 0, lane * 0))

    tmpv[...] = jnp.where(lane == 0, pos_tot,
                          jnp.where(lane == 1, neg_tot, 0))
    pltpu.sync_copy(tmpv, cnt_out.at[wid])

    @pl.when(wid == 0)
    def _():
        pxy[0, pl.ds(0, 16)] = lane.astype(jnp.float32) * 0.0
        pltpu.sync_copy(pxy.at[0, pl.ds(0, 16)], z_out)


@functools.partial(jax.jit, static_argnums=())
def _run_sc(boxes_t, gt_t, num_obj):
    mesh = plsc.VectorSubcoreMesh(core_axis_name="c", subcore_axis_name="s")
    kern = functools.partial(
        pl.kernel,
        mesh=mesh,
        compiler_params=pltpu.CompilerParams(needs_layout_passes=False),
        out_type=[
            jax.ShapeDtypeStruct((_NW, _ROW), jnp.int32),
            jax.ShapeDtypeStruct((_NW, _ROW), jnp.int32),
            jax.ShapeDtypeStruct((_NW, 16), jnp.int32),
            jax.ShapeDtypeStruct((16,), jnp.float32),
        ],
        scratch_types=[
            pltpu.VMEM((4, _CHUNK), jnp.float32),
            pltpu.VMEM((5, _CHUNK), jnp.float32),
            pltpu.VMEM((4, _M), jnp.float32),
            pltpu.VMEM((5, _M), jnp.float32),
            pltpu.VMEM((16,), jnp.int32),
            pltpu.VMEM((_BUFCAP + 16,), jnp.int32),
            pltpu.VMEM((_BUFCAP + 16,), jnp.int32),
            pltpu.VMEM((16,), jnp.int32),
        ],
    )(_sc_body)
    return kern(boxes_t, gt_t, num_obj)


def kernel(threshhold, batch_boxes, batch_classes, batch_gt, batch_num_objects):
    del threshhold, batch_classes
    boxes_p = jnp.pad(batch_boxes, ((0, 0), (0, _NPAD - _N), (0, 0)))
    boxes_t = jnp.transpose(boxes_p, (0, 2, 1))
    gt_t = jnp.transpose(batch_gt, (0, 2, 1))
    num_obj = jnp.broadcast_to(
        batch_num_objects.astype(jnp.int32)[:, None], (_B, 16))
    _pos, _neg, _cnt, z = _run_sc(boxes_t, gt_t, num_obj)
    return z[0]
